# baseline (speedup 1.0000x reference)
import functools

import jax
import jax.numpy as jnp
from jax import lax
from jax.experimental import pallas as pl
from jax.experimental.pallas import tpu as pltpu
from jax.experimental.pallas import tpu_sc as plsc

_B = 4096
_ND = 13
_NS = 26
_V = 100000
_D = 16
_REP = 64
_H1, _H2 = 256, 128
_NC, _NSUB = 2, 16
_NW = _NC * _NSUB
_BPW = _B // _NW
_NF = _NS // 2


def _sc_gather_body(idx_hbm, e2t_hbm, e1_hbm, xgt_out, e1gt_out,
                    idx_v, dst_v, e1dst_v, sem, sem_e1):
    w = lax.axis_index("s") * _NC + lax.axis_index("c")
    col0 = w * _BPW
    pltpu.sync_copy(idx_hbm.at[:, pl.ds(col0, _BPW)], idx_v)

    def per_field(f, carry):
        iv = idx_v.at[f]
        for d in range(_D):
            pltpu.async_copy(e2t_hbm.at[f, d].at[iv], dst_v.at[f * _D + d],
                             sem)
        pltpu.async_copy(e1_hbm.at[f].at[iv], e1dst_v.at[f], sem_e1)
        return carry

    lax.fori_loop(0, _NF, per_field, 0)
    pltpu.make_async_copy(xgt_out.at[:, pl.ds(col0, _BPW)], dst_v, sem).wait()
    pltpu.make_async_copy(e1gt_out.at[:, pl.ds(col0, _BPW)], e1dst_v,
                          sem_e1).wait()
    pltpu.sync_copy(dst_v, xgt_out.at[:, pl.ds(col0, _BPW)])
    pltpu.sync_copy(e1dst_v, e1gt_out.at[:, pl.ds(col0, _BPW)])


def _sc_gather(idx_t, e2t, e1):
    return pl.kernel(
        _sc_gather_body,
        out_type=(jax.ShapeDtypeStruct((_NF * _D, _B), jnp.float32),
                  jax.ShapeDtypeStruct((_NF, _B), jnp.float32)),
        mesh=plsc.VectorSubcoreMesh(core_axis_name="c", subcore_axis_name="s"),
        scratch_types=[pltpu.VMEM((_NF, _BPW), jnp.int32),
                       pltpu.VMEM((_NF * _D, _BPW), jnp.float32),
                       pltpu.VMEM((_NF, _BPW), jnp.float32),
                       pltpu.SemaphoreType.DMA,
                       pltpu.SemaphoreType.DMA],
        compiler_params=pltpu.CompilerParams(use_tc_tiling_on_sc=False),
    )(idx_t, e2t, e1)


_CBLK = 512


def _tc_body(xgta_ref, xgtb_ref, xdt_ref, rpt_ref, e1gta_ref, e1gtb_ref,
             w1ata_ref, w1atb_ref, w1bt_ref, w1ct_ref, bd1c_ref,
             wd2t_ref, bd2c_ref, wft_ref,
             w1dat_ref, w1dbt_ref, cb_ref,
             out_ref):
    f32 = jnp.float32

    def dot(a, b):
        return lax.dot_general(a, b, (((1,), (0,)), ((), ())),
                               preferred_element_type=f32)

    xga = xgta_ref[...]
    xgb = xgtb_ref[...]
    xd = xdt_ref[...]
    rp = rpt_ref[...]
    h1 = (dot(w1ata_ref[...], xga) + dot(w1atb_ref[...], xgb)
          + dot(w1bt_ref[...], xd) + dot(w1ct_ref[...], rp))
    h1 = jnp.maximum(h1 + bd1c_ref[...], 0.0)
    h2 = jnp.maximum(dot(wd2t_ref[...], h1) + bd2c_ref[...], 0.0)
    dnn = dot(wft_ref[...], h2)
    fm1d = dot(w1dat_ref[...], xd) + dot(w1dbt_ref[...], rp)
    r = lax.broadcasted_iota(jnp.int32, (_D, _NF * _D), 0)
    c = lax.broadcasted_iota(jnp.int32, (_D, _NF * _D), 1)
    m = ((c % _D) == r).astype(f32)
    s = dot(m, xga) + dot(m, xgb)
    ssq = dot(m, xga * xga) + dot(m, xgb * xgb)
    fm2 = 0.5 * jnp.sum(s * s - ssq, axis=0, keepdims=True)
    fm1s = (jnp.sum(e1gta_ref[...], axis=0, keepdims=True)
            + jnp.sum(e1gtb_ref[...], axis=0, keepdims=True))
    out_ref[...] = dnn + fm1d + fm2 + fm1s + cb_ref[...]


def _tc_dense(xgta, xgtb, xdt, rpt, e1gta, e1gtb, w1ata, w1atb, w1bt, w1ct,
              bd1c, wd2t, bd2c, wft, w1dat, w1dbt, cb):
    def blk(nrows):
        return pl.BlockSpec((nrows, _CBLK), lambda i: (0, i))

    def full(shape):
        return pl.BlockSpec(shape, lambda i: (0, 0))

    return pl.pallas_call(
        _tc_body,
        grid=(_B // _CBLK,),
        in_specs=[blk(_NF * _D), blk(_NF * _D), blk(_ND), blk(_REP),
                  blk(_NF), blk(_NF),
                  full((_H1, _NF * _D)), full((_H1, _NF * _D)),
                  full((_H1, _ND)), full((_H1, _REP)), full((_H1, 1)),
                  full((_H2, _H1)), full((_H2, 1)), full((1, _H2)),
                  full((1, _ND)), full((1, _REP)), full((1, 1))],
        out_specs=pl.BlockSpec((1, _CBLK), lambda i: (0, i)),
        out_shape=jax.ShapeDtypeStruct((1, _B), jnp.float32),
    )(xgta, xgtb, xdt, rpt, e1gta, e1gtb, w1ata, w1atb, w1bt, w1ct, bd1c,
      wd2t, bd2c, wft, w1dat, w1dbt, cb)


def kernel(representation, target_x, e1, e2, W1d, b1d, Wd1, bd1, Wd2, bd2, Wf, bf):
    txt = target_x.T
    idx_t = txt[_ND:].astype(jnp.int32)
    e2t = e2.transpose(0, 2, 1)
    xgta, e1gta = _sc_gather(idx_t[:_NF], e2t[:_NF], e1[:_NF])
    xgtb, e1gtb = _sc_gather(idx_t[_NF:], e2t[_NF:], e1[_NF:])
    w1a = Wd1[:_NS * _D]
    outt = _tc_dense(
        xgta, xgtb, txt[:_ND], representation.T, e1gta, e1gtb,
        w1a[:_NF * _D].T, w1a[_NF * _D:].T,
        Wd1[_NS * _D:_NS * _D + _ND].T, Wd1[_NS * _D + _ND:].T,
        bd1.reshape(_H1, 1), Wd2.T, bd2.reshape(_H2, 1), Wf.T,
        W1d[:_ND].T, W1d[_ND:].T, (b1d + bf).reshape(1, 1))
    return outt.reshape(_B, 1)


# speedup vs baseline: 1.2728x; 1.2728x over previous
import jax
import jax.numpy as jnp
from jax import lax
from jax.experimental import pallas as pl
from jax.experimental.pallas import tpu as pltpu
from jax.experimental.pallas import tpu_sc as plsc

_B = 4096
_ND = 13
_NS = 26
_V = 100000
_D = 16
_REP = 64
_H1, _H2 = 256, 128
_NC, _NSUB = 2, 16
_NW = _NC * _NSUB
_BPW = _B // _NW


def _sc_gather_body(idx_hbm, e2t_hbm, e1_hbm, xgt_out, e1gt_out,
                    idx_v, dst_v, e1dst_v, sem, sem_e1):
    w = lax.axis_index("s") * _NC + lax.axis_index("c")
    col0 = w * _BPW
    pltpu.sync_copy(idx_hbm.at[:, pl.ds(col0, _BPW)], idx_v)

    def per_field(f, carry):
        iv = idx_v.at[f]
        for d in range(_D):
            pltpu.async_copy(e2t_hbm.at[f, d].at[iv], dst_v.at[f * _D + d],
                             sem)
        pltpu.async_copy(e1_hbm.at[f].at[iv], e1dst_v.at[f], sem_e1)
        return carry

    lax.fori_loop(0, _NS, per_field, 0)
    pltpu.make_async_copy(xgt_out.at[:, pl.ds(col0, _BPW)], dst_v, sem).wait()
    pltpu.make_async_copy(e1gt_out.at[:, pl.ds(col0, _BPW)], e1dst_v,
                          sem_e1).wait()
    pltpu.sync_copy(dst_v, xgt_out.at[:, pl.ds(col0, _BPW)])
    pltpu.sync_copy(e1dst_v, e1gt_out.at[:, pl.ds(col0, _BPW)])


def _sc_gather(idx_t, e2t, e1):
    return pl.kernel(
        _sc_gather_body,
        out_type=(jax.ShapeDtypeStruct((_NS * _D, _B), jnp.float32),
                  jax.ShapeDtypeStruct((_NS, _B), jnp.float32)),
        mesh=plsc.VectorSubcoreMesh(core_axis_name="c", subcore_axis_name="s"),
        scratch_types=[pltpu.VMEM((_NS, _BPW), jnp.int32),
                       pltpu.VMEM((_NS * _D, _BPW), jnp.float32),
                       pltpu.VMEM((_NS, _BPW), jnp.float32),
                       pltpu.SemaphoreType.DMA,
                       pltpu.SemaphoreType.DMA],
        compiler_params=pltpu.CompilerParams(use_tc_tiling_on_sc=False),
    )(idx_t, e2t, e1)


_CBLK = 512


def _tc_body(xgt_ref, xdt_ref, rpt_ref, e1gt_ref,
             w1at_ref, w1bt_ref, w1ct_ref, bd1c_ref,
             wd2t_ref, bd2c_ref, wft_ref,
             w1dat_ref, w1dbt_ref, cb_ref,
             out_ref):
    f32 = jnp.float32

    def dot(a, b):
        return lax.dot_general(a, b, (((1,), (0,)), ((), ())),
                               preferred_element_type=f32)

    xg = xgt_ref[...]
    xd = xdt_ref[...]
    rp = rpt_ref[...]
    h1 = dot(w1at_ref[...], xg) + dot(w1bt_ref[...], xd) + dot(w1ct_ref[...], rp)
    h1 = jnp.maximum(h1 + bd1c_ref[...], 0.0)
    h2 = jnp.maximum(dot(wd2t_ref[...], h1) + bd2c_ref[...], 0.0)
    dnn = dot(wft_ref[...], h2)
    fm1d = dot(w1dat_ref[...], xd) + dot(w1dbt_ref[...], rp)
    r = lax.broadcasted_iota(jnp.int32, (_D, _NS * _D), 0)
    c = lax.broadcasted_iota(jnp.int32, (_D, _NS * _D), 1)
    m = ((c % _D) == r).astype(f32)
    s = dot(m, xg)
    ssq = dot(m, xg * xg)
    fm2 = 0.5 * jnp.sum(s * s - ssq, axis=0, keepdims=True)
    fm1s = jnp.sum(e1gt_ref[...], axis=0, keepdims=True)
    out_ref[...] = dnn + fm1d + fm2 + fm1s + cb_ref[...]


def _tc_dense(xgt, xdt, rpt, e1gt, w1at, w1bt, w1ct, bd1c, wd2t, bd2c, wft,
              w1dat, w1dbt, cb):
    def blk(nrows):
        return pl.BlockSpec((nrows, _CBLK), lambda i: (0, i))

    def full(shape):
        return pl.BlockSpec(shape, lambda i: (0, 0))

    return pl.pallas_call(
        _tc_body,
        grid=(_B // _CBLK,),
        in_specs=[blk(_NS * _D), blk(_ND), blk(_REP), blk(_NS),
                  full((_H1, _NS * _D)), full((_H1, _ND)), full((_H1, _REP)),
                  full((_H1, 1)),
                  full((_H2, _H1)), full((_H2, 1)), full((1, _H2)),
                  full((1, _ND)), full((1, _REP)), full((1, 1))],
        out_specs=pl.BlockSpec((1, _CBLK), lambda i: (0, i)),
        out_shape=jax.ShapeDtypeStruct((1, _B), jnp.float32),
    )(xgt, xdt, rpt, e1gt, w1at, w1bt, w1ct, bd1c, wd2t, bd2c, wft,
      w1dat, w1dbt, cb)


def kernel(representation, target_x, e1, e2, W1d, b1d, Wd1, bd1, Wd2, bd2, Wf, bf):
    txt = target_x.T
    idx_t = txt[_ND:].astype(jnp.int32)
    e2t = e2.transpose(0, 2, 1)
    xgt, e1gt = _sc_gather(idx_t, e2t, e1)
    outt = _tc_dense(
        xgt, txt[:_ND], representation.T, e1gt,
        Wd1[:_NS * _D].T, Wd1[_NS * _D:_NS * _D + _ND].T, Wd1[_NS * _D + _ND:].T,
        bd1.reshape(_H1, 1), Wd2.T, bd2.reshape(_H2, 1), Wf.T,
        W1d[:_ND].T, W1d[_ND:].T, (b1d + bf).reshape(1, 1))
    return outt.reshape(_B, 1)


# speedup vs baseline: 1.3205x; 1.0375x over previous
import jax
import jax.numpy as jnp
from jax import lax
from jax.experimental import pallas as pl
from jax.experimental.pallas import tpu as pltpu
from jax.experimental.pallas import tpu_sc as plsc

_B = 4096
_ND = 13
_NS = 26
_V = 100000
_D = 16
_REP = 64
_H1, _H2 = 256, 128
_NC, _NSUB = 2, 16
_NW = _NC * _NSUB
_BPW = _B // _NW


def _sc_e2_body(idx_hbm, e2t_hbm, xgt_out, idx_v, dst_v, sem):
    w = lax.axis_index("s") * _NC + lax.axis_index("c")
    col0 = w * _BPW
    pltpu.sync_copy(idx_hbm.at[:, pl.ds(col0, _BPW)], idx_v)

    def per_field(f, carry):
        iv = idx_v.at[f]
        for d in range(_D):
            pltpu.async_copy(e2t_hbm.at[f, d].at[iv], dst_v.at[f * _D + d],
                             sem)
        return carry

    lax.fori_loop(0, _NS, per_field, 0)
    pltpu.make_async_copy(xgt_out.at[:, pl.ds(col0, _BPW)], dst_v, sem).wait()
    pltpu.sync_copy(dst_v, xgt_out.at[:, pl.ds(col0, _BPW)])


def _sc_e2_gather(idx_t, e2t):
    return pl.kernel(
        _sc_e2_body,
        out_type=jax.ShapeDtypeStruct((_NS * _D, _B), jnp.float32),
        mesh=plsc.VectorSubcoreMesh(core_axis_name="c", subcore_axis_name="s"),
        scratch_types=[pltpu.VMEM((_NS, _BPW), jnp.int32),
                       pltpu.VMEM((_NS * _D, _BPW), jnp.float32),
                       pltpu.SemaphoreType.DMA],
        compiler_params=pltpu.CompilerParams(use_tc_tiling_on_sc=False),
    )(idx_t, e2t)


def _sc_e1_body(idx_hbm, e1_hbm, e1gt_out, idx_v, e1dst_v, sem):
    w = lax.axis_index("s") * _NC + lax.axis_index("c")
    col0 = w * _BPW
    pltpu.sync_copy(idx_hbm.at[:, pl.ds(col0, _BPW)], idx_v)

    def per_field(f, carry):
        pltpu.async_copy(e1_hbm.at[f].at[idx_v.at[f]], e1dst_v.at[f], sem)
        return carry

    lax.fori_loop(0, _NS, per_field, 0)
    pltpu.make_async_copy(e1gt_out.at[:, pl.ds(col0, _BPW)], e1dst_v,
                          sem).wait()
    pltpu.sync_copy(e1dst_v, e1gt_out.at[:, pl.ds(col0, _BPW)])


def _sc_e1_gather(idx_t, e1):
    return pl.kernel(
        _sc_e1_body,
        out_type=jax.ShapeDtypeStruct((_NS, _B), jnp.float32),
        mesh=plsc.VectorSubcoreMesh(core_axis_name="c", subcore_axis_name="s"),
        scratch_types=[pltpu.VMEM((_NS, _BPW), jnp.int32),
                       pltpu.VMEM((_NS, _BPW), jnp.float32),
                       pltpu.SemaphoreType.DMA],
        compiler_params=pltpu.CompilerParams(use_tc_tiling_on_sc=False),
    )(idx_t, e1)


_CBLK = 512


def _tc_body(xgt_ref, xdt_ref, rpt_ref, e1gt_ref,
             w1at_ref, w1bt_ref, w1ct_ref, bd1c_ref,
             wd2t_ref, bd2c_ref, wft_ref,
             w1dat_ref, w1dbt_ref, cb_ref,
             out_ref):
    f32 = jnp.float32

    def dot(a, b):
        return lax.dot_general(a, b, (((1,), (0,)), ((), ())),
                               preferred_element_type=f32)

    xg = xgt_ref[...]
    xd = xdt_ref[...]
    rp = rpt_ref[...]
    h1 = dot(w1at_ref[...], xg) + dot(w1bt_ref[...], xd) + dot(w1ct_ref[...], rp)
    h1 = jnp.maximum(h1 + bd1c_ref[...], 0.0)
    h2 = jnp.maximum(dot(wd2t_ref[...], h1) + bd2c_ref[...], 0.0)
    dnn = dot(wft_ref[...], h2)
    fm1d = dot(w1dat_ref[...], xd) + dot(w1dbt_ref[...], rp)
    r = lax.broadcasted_iota(jnp.int32, (_D, _NS * _D), 0)
    c = lax.broadcasted_iota(jnp.int32, (_D, _NS * _D), 1)
    m = ((c % _D) == r).astype(f32)
    s = dot(m, xg)
    ssq = dot(m, xg * xg)
    fm2 = 0.5 * jnp.sum(s * s - ssq, axis=0, keepdims=True)
    fm1s = jnp.sum(e1gt_ref[...], axis=0, keepdims=True)
    out_ref[...] = dnn + fm1d + fm2 + fm1s + cb_ref[...]


def _tc_dense(xgt, xdt, rpt, e1gt, w1at, w1bt, w1ct, bd1c, wd2t, bd2c, wft,
              w1dat, w1dbt, cb):
    def blk(nrows):
        return pl.BlockSpec((nrows, _CBLK), lambda i: (0, i))

    def full(shape):
        return pl.BlockSpec(shape, lambda i: (0, 0))

    return pl.pallas_call(
        _tc_body,
        grid=(_B // _CBLK,),
        in_specs=[blk(_NS * _D), blk(_ND), blk(_REP), blk(_NS),
                  full((_H1, _NS * _D)), full((_H1, _ND)), full((_H1, _REP)),
                  full((_H1, 1)),
                  full((_H2, _H1)), full((_H2, 1)), full((1, _H2)),
                  full((1, _ND)), full((1, _REP)), full((1, 1))],
        out_specs=pl.BlockSpec((1, _CBLK), lambda i: (0, i)),
        out_shape=jax.ShapeDtypeStruct((1, _B), jnp.float32),
    )(xgt, xdt, rpt, e1gt, w1at, w1bt, w1ct, bd1c, wd2t, bd2c, wft,
      w1dat, w1dbt, cb)


def kernel(representation, target_x, e1, e2, W1d, b1d, Wd1, bd1, Wd2, bd2, Wf, bf):
    txt = target_x.T
    idx_t = txt[_ND:].astype(jnp.int32)
    e2t = e2.transpose(0, 2, 1)
    e1gt = _sc_e1_gather(idx_t, e1)
    xgt = _sc_e2_gather(idx_t, e2t)
    outt = _tc_dense(
        xgt, txt[:_ND], representation.T, e1gt,
        Wd1[:_NS * _D].T, Wd1[_NS * _D:_NS * _D + _ND].T, Wd1[_NS * _D + _ND:].T,
        bd1.reshape(_H1, 1), Wd2.T, bd2.reshape(_H2, 1), Wf.T,
        W1d[:_ND].T, W1d[_ND:].T, (b1d + bf).reshape(1, 1))
    return outt.reshape(_B, 1)


# speedup vs baseline: 1.3208x; 1.0002x over previous
import jax
import jax.numpy as jnp
from jax import lax
from jax.experimental import pallas as pl
from jax.experimental.pallas import tpu as pltpu
from jax.experimental.pallas import tpu_sc as plsc

_B = 4096
_ND = 13
_NS = 26
_V = 100000
_D = 16
_REP = 64
_H1, _H2 = 256, 128
_NC, _NSUB = 2, 16
_NW = _NC * _NSUB
_BPW = _B // _NW


def _sc_e2_body(idx_hbm, e2t_hbm, xgt_out, idx_v, dst_v, sem_a, sem_b):
    w = lax.axis_index("s") * _NC + lax.axis_index("c")
    col0 = w * _BPW
    pltpu.sync_copy(idx_hbm.at[:, pl.ds(col0, _BPW)], idx_v)

    def per_field(f, carry):
        iv = idx_v.at[f]
        for d in range(_D):
            pltpu.async_copy(e2t_hbm.at[f, d].at[iv], dst_v.at[f * _D + d],
                             sem_a if d % 2 == 0 else sem_b)
        return carry

    lax.fori_loop(0, _NS, per_field, 0)
    half = _NS * _D // 2
    pltpu.make_async_copy(xgt_out.at[pl.ds(0, half), pl.ds(col0, _BPW)],
                          dst_v.at[pl.ds(0, half)], sem_a).wait()
    pltpu.make_async_copy(xgt_out.at[pl.ds(0, half), pl.ds(col0, _BPW)],
                          dst_v.at[pl.ds(0, half)], sem_b).wait()
    pltpu.sync_copy(dst_v, xgt_out.at[:, pl.ds(col0, _BPW)])


def _sc_e2_gather(idx_t, e2t):
    return pl.kernel(
        _sc_e2_body,
        out_type=jax.ShapeDtypeStruct((_NS * _D, _B), jnp.float32),
        mesh=plsc.VectorSubcoreMesh(core_axis_name="c", subcore_axis_name="s"),
        scratch_types=[pltpu.VMEM((_NS, _BPW), jnp.int32),
                       pltpu.VMEM((_NS * _D, _BPW), jnp.float32),
                       pltpu.SemaphoreType.DMA,
                       pltpu.SemaphoreType.DMA],
        compiler_params=pltpu.CompilerParams(use_tc_tiling_on_sc=False),
    )(idx_t, e2t)


def _sc_e1_body(idx_hbm, e1_hbm, e1gt_out, idx_v, e1dst_v, sem):
    w = lax.axis_index("s") * _NC + lax.axis_index("c")
    col0 = w * _BPW
    pltpu.sync_copy(idx_hbm.at[:, pl.ds(col0, _BPW)], idx_v)

    def per_field(f, carry):
        pltpu.async_copy(e1_hbm.at[f].at[idx_v.at[f]], e1dst_v.at[f], sem)
        return carry

    lax.fori_loop(0, _NS, per_field, 0)
    pltpu.make_async_copy(e1gt_out.at[:, pl.ds(col0, _BPW)], e1dst_v,
                          sem).wait()
    pltpu.sync_copy(e1dst_v, e1gt_out.at[:, pl.ds(col0, _BPW)])


def _sc_e1_gather(idx_t, e1):
    return pl.kernel(
        _sc_e1_body,
        out_type=jax.ShapeDtypeStruct((_NS, _B), jnp.float32),
        mesh=plsc.VectorSubcoreMesh(core_axis_name="c", subcore_axis_name="s"),
        scratch_types=[pltpu.VMEM((_NS, _BPW), jnp.int32),
                       pltpu.VMEM((_NS, _BPW), jnp.float32),
                       pltpu.SemaphoreType.DMA],
        compiler_params=pltpu.CompilerParams(use_tc_tiling_on_sc=False),
    )(idx_t, e1)


_CBLK = 512


def _tc_body(xgt_ref, xdt_ref, rpt_ref, e1gt_ref,
             w1at_ref, w1bt_ref, w1ct_ref, bd1c_ref,
             wd2t_ref, bd2c_ref, wft_ref,
             w1dat_ref, w1dbt_ref, cb_ref,
             out_ref):
    f32 = jnp.float32

    def dot(a, b):
        return lax.dot_general(a, b, (((1,), (0,)), ((), ())),
                               preferred_element_type=f32)

    xg = xgt_ref[...]
    xd = xdt_ref[...]
    rp = rpt_ref[...]
    h1 = dot(w1at_ref[...], xg) + dot(w1bt_ref[...], xd) + dot(w1ct_ref[...], rp)
    h1 = jnp.maximum(h1 + bd1c_ref[...], 0.0)
    h2 = jnp.maximum(dot(wd2t_ref[...], h1) + bd2c_ref[...], 0.0)
    dnn = dot(wft_ref[...], h2)
    fm1d = dot(w1dat_ref[...], xd) + dot(w1dbt_ref[...], rp)
    r = lax.broadcasted_iota(jnp.int32, (_D, _NS * _D), 0)
    c = lax.broadcasted_iota(jnp.int32, (_D, _NS * _D), 1)
    m = ((c % _D) == r).astype(f32)
    s = dot(m, xg)
    ssq = dot(m, xg * xg)
    fm2 = 0.5 * jnp.sum(s * s - ssq, axis=0, keepdims=True)
    fm1s = jnp.sum(e1gt_ref[...], axis=0, keepdims=True)
    out_ref[...] = dnn + fm1d + fm2 + fm1s + cb_ref[...]


def _tc_dense(xgt, xdt, rpt, e1gt, w1at, w1bt, w1ct, bd1c, wd2t, bd2c, wft,
              w1dat, w1dbt, cb):
    def blk(nrows):
        return pl.BlockSpec((nrows, _CBLK), lambda i: (0, i))

    def full(shape):
        return pl.BlockSpec(shape, lambda i: (0, 0))

    return pl.pallas_call(
        _tc_body,
        grid=(_B // _CBLK,),
        in_specs=[blk(_NS * _D), blk(_ND), blk(_REP), blk(_NS),
                  full((_H1, _NS * _D)), full((_H1, _ND)), full((_H1, _REP)),
                  full((_H1, 1)),
                  full((_H2, _H1)), full((_H2, 1)), full((1, _H2)),
                  full((1, _ND)), full((1, _REP)), full((1, 1))],
        out_specs=pl.BlockSpec((1, _CBLK), lambda i: (0, i)),
        out_shape=jax.ShapeDtypeStruct((1, _B), jnp.float32),
    )(xgt, xdt, rpt, e1gt, w1at, w1bt, w1ct, bd1c, wd2t, bd2c, wft,
      w1dat, w1dbt, cb)


def kernel(representation, target_x, e1, e2, W1d, b1d, Wd1, bd1, Wd2, bd2, Wf, bf):
    txt = target_x.T
    idx_t = txt[_ND:].astype(jnp.int32)
    e2t = e2.transpose(0, 2, 1)
    e1gt = _sc_e1_gather(idx_t, e1)
    xgt = _sc_e2_gather(idx_t, e2t)
    outt = _tc_dense(
        xgt, txt[:_ND], representation.T, e1gt,
        Wd1[:_NS * _D].T, Wd1[_NS * _D:_NS * _D + _ND].T, Wd1[_NS * _D + _ND:].T,
        bd1.reshape(_H1, 1), Wd2.T, bd2.reshape(_H2, 1), Wf.T,
        W1d[:_ND].T, W1d[_ND:].T, (b1d + bf).reshape(1, 1))
    return outt.reshape(_B, 1)
